# baseline (speedup 1.0000x reference)
import functools

import jax
import jax.numpy as jnp
import numpy as np
from jax import lax
from jax.experimental import pallas as pl
from jax.experimental.pallas import tpu as pltpu
from jax.experimental.pallas import tpu_sc as plsc

_N = 1024
_E = 4096
_NVEC = 128
_PROP = 2
_M1 = 11
_EPT = _E // 16
_GRP = _EPT // 16


def _shift_mats():
    H = np.arange(15); p = np.arange(8); a = np.arange(8)
    m1 = (p[None, :, None] + a[None, None, :] == H[:, None, None])
    r = np.arange(8); A = np.arange(22)
    cr = (A[None, :, None] - r[:, None, None] == H[None, None, :])
    t = np.arange(4); B = np.arange(12); Q = np.arange(9)
    ct = (B[None, :, None] - t[:, None, None] == Q[None, None, :])
    oh = np.arange(3); ih = np.arange(8)
    ph = (A[None, None, :] == ih[None, :, None] + 8 - oh[:, None, None])
    ow = np.arange(5); iw = np.arange(16)
    pw = (B[None, None, :] == iw[None, :, None] - ow[:, None, None])
    f32 = np.float32
    return m1.astype(f32), cr.astype(f32), ct.astype(f32), ph.astype(f32), pw.astype(f32)

_M1S, _CR, _CT, _PH, _PW = _shift_mats()


def _fold_stacks(ps, nch):
    st = lambda k: jnp.stack([p[k] for p in ps])
    c1w = st("c1w")[:, :, :, :, 0]
    c2w, c2b = st("c2w"), st("c2b")
    c3w, c3b = st("c3w"), st("c3b")
    lw, lb = st("lw"), st("lb")
    c1b = st("c1b")
    T = jnp.einsum("nompq,nmca->nocpaq", c2w, c1w)
    K12 = jnp.einsum("nocpaq,Hpa->nocHq", T, _M1S)
    Z = jnp.einsum("nuort,nocHq->nucrtHq", c3w, K12)
    Z1 = jnp.einsum("nucrtHq,rAH->nuctqA", Z, _CR)
    K123 = jnp.einsum("nuctqA,tBq->nucAB", Z1, _CT)
    Y = jnp.einsum("nucAB,hgA->nucBhg", K123, _PH)
    Af = jnp.einsum("nucBhg,wjB->nuhwcgj", Y, _PW)
    Af = Af.reshape(Af.shape[0], 75, nch * 128)
    A = jnp.einsum("nkf,nfm->nkm", lw, Af)
    b2 = c2b + jnp.einsum("nom,nm->no", c2w.sum((3, 4)), c1b)
    b3 = c3b + jnp.einsum("nuo,no->nu", c3w.sum((3, 4)), b2)
    b = lb + jnp.einsum("nkf,nf->nk", lw, jnp.repeat(b3, 15, axis=1))
    return A, b


def _fold_all(params):
    mps = [params["av"]["fwd"], params["av"]["bwd"],
           params["pol"]["fwd"], params["pol"]["bwd"]]
    A4, bh4 = _fold_stacks(mps, 2)
    ndw = jnp.stack([p["ndw"] for p in mps]).reshape(4, 128, 65)
    ndb = jnp.stack([p["ndb"] for p in mps]).reshape(4, 128)
    M1m = ndw[:, :, :64]
    wa = ndw[:, :, 64]
    G = jnp.einsum("nkf,npk->nfp", A4.reshape(4, 64, 2, 128).transpose(
        0, 2, 1, 3).reshape(8, 64, 128), jnp.repeat(M1m, 2, axis=0))
    q0 = jnp.einsum("npk,nk->np", M1m, bh4) + ndb
    An2, bn2 = _fold_stacks([params["av"], params["pol"]], 1)

    def rep(i):
        gp = params["av"] if i == 0 else params["pol"]
        o = 2 * i
        return (G[2 * o], G[2 * o + 1], G[2 * o + 2], G[2 * o + 3],
                q0[o][None, :], wa[o][None, :], q0[o + 1][None, :], wa[o + 1][None, :],
                An2[i].T, bn2[i][None, :], gp["gmw"].T, gp["gmb"][None, :],
                gp["fmw"].T, gp["fmb"][None, :])

    return rep(0), rep(1)


def _sc_body(src_hbm, dst_hbm, ea_hbm, zeros_hbm,
             w_out, sad_out, sas_out,
             src_v, dst_v, ea_v, oh_w, oh_a, zbuf, wsh, sash):
    c = lax.axis_index("c")
    s = lax.axis_index("s")
    is_w = c == 0

    pltpu.sync_copy(zeros_hbm.at[pl.ds(0, 1024)], zbuf)
    for b in range(2):
        pltpu.sync_copy(zbuf, wsh.at[pl.ds(s * 2048 + b * 1024, 1024)])

    @pl.when(s == 0)
    def _():
        pltpu.sync_copy(zeros_hbm.at[pl.ds(0, 64)], sash)

    base = s * _EPT
    pltpu.sync_copy(src_hbm.at[pl.ds(base, _EPT)], src_v)
    pltpu.sync_copy(dst_hbm.at[pl.ds(base, _EPT)], dst_v)
    pltpu.sync_copy(ea_hbm.at[pl.ds(base, _EPT)], ea_v)

    zvec = jnp.zeros((16,), jnp.float32)
    for i in range(16):
        oh_w[i, :] = zvec
        oh_a[i, :] = zvec

    plsc.subcore_barrier()

    iota = lax.iota(jnp.int32, 16)
    ones = jnp.ones((16,), jnp.float32)
    rbase = c * 32768
    for g in range(_GRP):
        d = dst_v[pl.ds(g * 16, 16)]
        sr = src_v[pl.ds(g * 16, 16)]
        ev = ea_v[pl.ds(g * 16, 16)]
        f = d * 1024 + sr
        r = lax.shift_right_logical(f, 4) - rbase
        ok = jnp.logical_and(r >= 0, r < 32768)
        r = jnp.where(ok, r, 32768)
        l = lax.bitwise_and(f, 15)
        plsc.store_scatter(oh_w, [iota, l], ones)
        pltpu.sync_copy(oh_w, wsh.at[r], add=True)
        plsc.store_scatter(oh_w, [iota, l], zvec)
        a = jnp.where(is_w, d, sr)
        r2 = lax.shift_right_logical(a, 4)
        l2 = lax.bitwise_and(a, 15)
        plsc.store_scatter(oh_a, [iota, l2], ev)
        pltpu.sync_copy(oh_a, sash.at[r2], add=True)
        plsc.store_scatter(oh_a, [iota, l2], zvec)

    plsc.subcore_barrier()

    for b in range(2):
        pltpu.sync_copy(wsh.at[pl.ds(s * 2048 + b * 1024, 1024)], zbuf)
        pltpu.sync_copy(zbuf, w_out.at[pl.ds(rbase + s * 2048 + b * 1024, 1024)])

    @pl.when(jnp.logical_and(is_w, s == 0))
    def _():
        pltpu.sync_copy(sash, sad_out)

    @pl.when(jnp.logical_and(jnp.logical_not(is_w), s == 0))
    def _():
        pltpu.sync_copy(sash, sas_out)


@functools.lru_cache(maxsize=None)
def _sc_build():
    return pl.kernel(
        _sc_body,
        out_type=(
            jax.ShapeDtypeStruct((65536, 16), jnp.float32),
            jax.ShapeDtypeStruct((64, 16), jnp.float32),
            jax.ShapeDtypeStruct((64, 16), jnp.float32),
        ),
        mesh=plsc.VectorSubcoreMesh(core_axis_name="c", subcore_axis_name="s"),
        compiler_params=pltpu.CompilerParams(needs_layout_passes=False,
                                             use_tc_tiling_on_sc=False),
        scratch_types=(
            pltpu.VMEM((_EPT,), jnp.int32),
            pltpu.VMEM((_EPT,), jnp.int32),
            pltpu.VMEM((_EPT,), jnp.float32),
            pltpu.VMEM((16, 16), jnp.float32),
            pltpu.VMEM((16, 16), jnp.float32),
            pltpu.VMEM((1024, 16), jnp.float32),
            pltpu.VMEM_SHARED((32784, 16), jnp.float32),
            pltpu.VMEM_SHARED((64, 16), jnp.float32),
        ),
    )


def _sigmoid(x):
    t = jnp.exp(-jnp.abs(x))
    return jnp.where(x >= 0.0, 1.0 / (1.0 + t), t / (1.0 + t))


def _tc_body(x0, w, sad, sas,
             a_Gfi, a_Gfj, a_Gbi, a_Gbj, a_q0f, a_waf, a_q0b, a_wab,
             a_AnT, a_bn, a_gmwT, a_gmb, a_fmwT, a_fmb,
             p_Gfi, p_Gfj, p_Gbi, p_Gbj, p_q0f, p_waf, p_q0b, p_wab,
             p_AnT, p_bn, p_gmwT, p_gmb, p_fmwT, p_fmb,
             pwat, pwk, pbr, qwh, qwk, qwx, qb,
             q_out, k_out, xk_out):
    f32 = jnp.float32
    bf16 = jnp.bfloat16
    W = w[:, :]
    X0 = x0[:, :]
    sa_d = sad[:, :]
    sa_s = sas[:, :]

    def dot(a, b):
        return jnp.dot(a, b, preferred_element_type=f32)

    def tdot(a, b):
        return lax.dot_general(a, b, (((0,), (0,)), ((), ())),
                               preferred_element_type=f32)

    ones_col = jnp.ones((_N, 1), f32)
    deg_d = dot(W, ones_col)
    deg_s = tdot(W, ones_col)

    Wb = W.astype(bf16)

    def wdots(X):
        xh = X.astype(bf16)
        xl = (X - xh.astype(f32)).astype(bf16)
        wx = dot(Wb, xh) + dot(Wb, xl)
        wtx = tdot(Wb, xh) + tdot(Wb, xl)
        return wx, wtx

    def rep(Gfi, Gfj, Gbi, Gbj, q0f, waf, q0b, wab, AnT, bn, gmwT, gmb, fmwT, fmb):
        R = (deg_d * q0f[:, :] + sa_d * waf[:, :]
             + deg_s * q0b[:, :] + sa_s * wab[:, :])
        X = X0
        for _ in range(_PROP):
            wx, wtx = wdots(X)
            X = (deg_d * dot(X, Gfi[:, :]) + deg_s * dot(X, Gbi[:, :])
                 + dot(wx, Gfj[:, :]) + dot(wtx, Gbj[:, :]) + R)
        h = dot(X, AnT[:, :]) + bn[:, :]
        g = _sigmoid(dot(h, gmwT[:, :]) + gmb[:, :])
        hv = dot(h, fmwT[:, :]) + fmb[:, :]
        return jnp.sum(g * hv, axis=0, keepdims=True)

    h_av = rep(a_Gfi, a_Gfj, a_Gbi, a_Gbj, a_q0f, a_waf, a_q0b, a_wab,
               a_AnT, a_bn, a_gmwT, a_gmb, a_fmwT, a_fmb)
    h_pol = rep(p_Gfi, p_Gfj, p_Gbi, p_Gbj, p_q0f, p_waf, p_q0b, p_wab,
                p_AnT, p_bn, p_gmwT, p_gmb, p_fmwT, p_fmb)

    kint = lax.broadcasted_iota(jnp.int32, (16, 1), 0)
    kcol = kint.astype(f32)
    row0 = dot(h_pol, pwat[:, :]) + pbr[:, :]
    z = 0.1 * (row0 + kcol * pwk[:, :])
    xk = (jnp.maximum(z, 0.0) + jnp.log(1.0 + jnp.exp(-jnp.abs(z)))) * 10.0
    q_base = dot(h_av, qwh[:, :])[0, 0] + qb[0, 0]
    Q = q_base + kcol * qwk[0, 0] + dot(xk, qwx[:, :])
    Qm = jnp.where(kint < _M1, Q, -1e30)
    qmax = jnp.max(Qm, keepdims=True)
    kstar = jnp.min(jnp.where(Qm == qmax, kint, 2**30), keepdims=True)
    q_out[:, :] = qmax
    k_out[:, :] = kstar
    sel = (kint == kstar).astype(f32)
    xk_out[:, :] = jnp.sum(xk * sel, axis=0, keepdims=True)


def _tc_forward(X0, W, sad, sas, fa, fp, head, interpret=False):
    out = pl.pallas_call(
        _tc_body,
        out_shape=(
            jax.ShapeDtypeStruct((1, 1), jnp.float32),
            jax.ShapeDtypeStruct((1, 1), jnp.int32),
            jax.ShapeDtypeStruct((1, 128), jnp.float32),
        ),
        interpret=interpret,
    )(X0, W, sad, sas, *fa, *fp, *head)
    return out


def _fold_head(params):
    pw = params["pw"].reshape(128, 51)
    qw = params["qw"][0]
    return (pw[:, :50].T, pw[:, 50][None, :], params["pb"].reshape(1, 128),
            qw[:50][:, None], qw[50].reshape(1, 1), qw[51:][:, None],
            params["qb"].reshape(1, 1))


def kernel(x, edge_attr, params, edge_index, done):
    src = edge_index[0]
    dst = edge_index[1]
    zeros = jnp.zeros((1024, 16), jnp.float32)
    X0 = x.reshape(_N, _NVEC)
    fa, fp = _fold_all(params)
    head = _fold_head(params)
    Wr, sad_r, sas_r = _sc_build()(src, dst, edge_attr, zeros)
    W = Wr.reshape(1024, 1024)
    sad = sad_r.reshape(1024, 1)
    sas = sas_r.reshape(1024, 1)
    q, k, xkrow = _tc_forward(X0, W, sad, sas, fa, fp, head)
    return q.reshape(()), k.reshape(()), xkrow.reshape(8, 16)


# speedup vs baseline: 1.0028x; 1.0028x over previous
import functools

import jax
import jax.numpy as jnp
import numpy as np
from jax import lax
from jax.experimental import pallas as pl
from jax.experimental.pallas import tpu as pltpu
from jax.experimental.pallas import tpu_sc as plsc

_N = 1024
_E = 4096
_NVEC = 128
_PROP = 2
_M1 = 11
_EPT = _E // 16
_GRP = _EPT // 16


def _shift_mats():
    H = np.arange(15); p = np.arange(8); a = np.arange(8)
    m1 = (p[None, :, None] + a[None, None, :] == H[:, None, None])
    r = np.arange(8); A = np.arange(22)
    cr = (A[None, :, None] - r[:, None, None] == H[None, None, :])
    t = np.arange(4); B = np.arange(12); Q = np.arange(9)
    ct = (B[None, :, None] - t[:, None, None] == Q[None, None, :])
    oh = np.arange(3); ih = np.arange(8)
    ph = (A[None, None, :] == ih[None, :, None] + 8 - oh[:, None, None])
    ow = np.arange(5); iw = np.arange(16)
    pw = (B[None, None, :] == iw[None, :, None] - ow[:, None, None])
    f32 = np.float32
    return m1.astype(f32), cr.astype(f32), ct.astype(f32), ph.astype(f32), pw.astype(f32)

_M1S, _CR, _CT, _PH, _PW = _shift_mats()


def _fold_stacks(ps, nch):
    st = lambda k: jnp.stack([p[k] for p in ps])
    c1w = st("c1w")[:, :, :, :, 0]
    c2w, c2b = st("c2w"), st("c2b")
    c3w, c3b = st("c3w"), st("c3b")
    lw, lb = st("lw"), st("lb")
    c1b = st("c1b")
    T = jnp.einsum("nompq,nmca->nocpaq", c2w, c1w)
    K12 = jnp.einsum("nocpaq,Hpa->nocHq", T, _M1S)
    Z = jnp.einsum("nuort,nocHq->nucrtHq", c3w, K12)
    Z1 = jnp.einsum("nucrtHq,rAH->nuctqA", Z, _CR)
    K123 = jnp.einsum("nuctqA,tBq->nucAB", Z1, _CT)
    Y = jnp.einsum("nucAB,hgA->nucBhg", K123, _PH)
    Af = jnp.einsum("nucBhg,wjB->nuhwcgj", Y, _PW)
    Af = Af.reshape(Af.shape[0], 75, nch * 128)
    A = jnp.einsum("nkf,nfm->nkm", lw, Af)
    b2 = c2b + jnp.einsum("nom,nm->no", c2w.sum((3, 4)), c1b)
    b3 = c3b + jnp.einsum("nuo,no->nu", c3w.sum((3, 4)), b2)
    b = lb + jnp.einsum("nkf,nf->nk", lw, jnp.repeat(b3, 15, axis=1))
    return A, b


def _fold_all(params):
    mps = [params["av"]["fwd"], params["av"]["bwd"],
           params["pol"]["fwd"], params["pol"]["bwd"]]
    A4, bh4 = _fold_stacks(mps, 2)
    ndw = jnp.stack([p["ndw"] for p in mps]).reshape(4, 128, 65)
    ndb = jnp.stack([p["ndb"] for p in mps]).reshape(4, 128)
    M1m = ndw[:, :, :64]
    wa = ndw[:, :, 64]
    G = jnp.einsum("nkf,npk->nfp", A4.reshape(4, 64, 2, 128).transpose(
        0, 2, 1, 3).reshape(8, 64, 128), jnp.repeat(M1m, 2, axis=0))
    q0 = jnp.einsum("npk,nk->np", M1m, bh4) + ndb
    An2, bn2 = _fold_stacks([params["av"], params["pol"]], 1)

    def rep(i):
        gp = params["av"] if i == 0 else params["pol"]
        o = 2 * i
        return (G[2 * o], G[2 * o + 1], G[2 * o + 2], G[2 * o + 3],
                q0[o][None, :], wa[o][None, :], q0[o + 1][None, :], wa[o + 1][None, :],
                An2[i].T, bn2[i][None, :], gp["gmw"].T, gp["gmb"][None, :],
                gp["fmw"].T, gp["fmb"][None, :])

    return rep(0), rep(1)


def _sc_body(src_hbm, dst_hbm, ea_hbm, zeros_hbm,
             w_out, sad_out, sas_out,
             src_v, dst_v, ea_v, oh_w, oh_a, zbuf, wsh, sash):
    c = lax.axis_index("c")
    s = lax.axis_index("s")
    is_w = c == 0

    pltpu.sync_copy(zeros_hbm.at[pl.ds(0, 1024)], zbuf)
    for b in range(2):
        pltpu.sync_copy(zbuf, wsh.at[pl.ds(s * 2048 + b * 1024, 1024)])

    @pl.when(s == 0)
    def _():
        pltpu.sync_copy(zeros_hbm.at[pl.ds(0, 64)], sash)

    base = s * _EPT
    pltpu.sync_copy(src_hbm.at[pl.ds(base, _EPT)], src_v)
    pltpu.sync_copy(dst_hbm.at[pl.ds(base, _EPT)], dst_v)
    pltpu.sync_copy(ea_hbm.at[pl.ds(base, _EPT)], ea_v)

    zvec = jnp.zeros((16,), jnp.float32)
    for i in range(16):
        oh_w[i, :] = zvec
        oh_a[i, :] = zvec

    plsc.subcore_barrier()

    iota = lax.iota(jnp.int32, 16)
    ones = jnp.ones((16,), jnp.float32)
    rbase = c * 32768
    for g in range(_GRP):
        d = dst_v[pl.ds(g * 16, 16)]
        sr = src_v[pl.ds(g * 16, 16)]
        ev = ea_v[pl.ds(g * 16, 16)]
        f = d * 1024 + sr
        r = lax.shift_right_logical(f, 4) - rbase
        ok = jnp.logical_and(r >= 0, r < 32768)
        r = jnp.where(ok, r, 32768)
        l = lax.bitwise_and(f, 15)
        plsc.store_scatter(oh_w, [iota, l], ones)
        pltpu.sync_copy(oh_w, wsh.at[r], add=True)
        plsc.store_scatter(oh_w, [iota, l], zvec)
        a = jnp.where(is_w, d, sr)
        r2 = lax.shift_right_logical(a, 4)
        l2 = lax.bitwise_and(a, 15)
        plsc.store_scatter(oh_a, [iota, l2], ev)
        pltpu.sync_copy(oh_a, sash.at[r2], add=True)
        plsc.store_scatter(oh_a, [iota, l2], zvec)

    plsc.subcore_barrier()

    for b in range(2):
        pltpu.sync_copy(wsh.at[pl.ds(s * 2048 + b * 1024, 1024)], zbuf)
        pltpu.sync_copy(zbuf, w_out.at[pl.ds(rbase + s * 2048 + b * 1024, 1024)])

    @pl.when(jnp.logical_and(is_w, s == 0))
    def _():
        pltpu.sync_copy(sash, sad_out)

    @pl.when(jnp.logical_and(jnp.logical_not(is_w), s == 0))
    def _():
        pltpu.sync_copy(sash, sas_out)


@functools.lru_cache(maxsize=None)
def _sc_build():
    return pl.kernel(
        _sc_body,
        out_type=(
            jax.ShapeDtypeStruct((65536, 16), jnp.float32),
            jax.ShapeDtypeStruct((64, 16), jnp.float32),
            jax.ShapeDtypeStruct((64, 16), jnp.float32),
        ),
        mesh=plsc.VectorSubcoreMesh(core_axis_name="c", subcore_axis_name="s"),
        compiler_params=pltpu.CompilerParams(needs_layout_passes=False,
                                             use_tc_tiling_on_sc=False),
        scratch_types=(
            pltpu.VMEM((_EPT,), jnp.int32),
            pltpu.VMEM((_EPT,), jnp.int32),
            pltpu.VMEM((_EPT,), jnp.float32),
            pltpu.VMEM((16, 16), jnp.float32),
            pltpu.VMEM((16, 16), jnp.float32),
            pltpu.VMEM((1024, 16), jnp.float32),
            pltpu.VMEM_SHARED((32784, 16), jnp.float32),
            pltpu.VMEM_SHARED((64, 16), jnp.float32),
        ),
    )


def _sigmoid(x):
    t = jnp.exp(-jnp.abs(x))
    return jnp.where(x >= 0.0, 1.0 / (1.0 + t), t / (1.0 + t))


def _tc_body(x0, w, sad, sas,
             a_Gfi, a_Gfj, a_Gbi, a_Gbj, a_q0f, a_waf, a_q0b, a_wab,
             a_AnT, a_bn, a_gmwT, a_gmb, a_fmwT, a_fmb,
             p_Gfi, p_Gfj, p_Gbi, p_Gbj, p_q0f, p_waf, p_q0b, p_wab,
             p_AnT, p_bn, p_gmwT, p_gmb, p_fmwT, p_fmb,
             pwat, pwk, pbr, qwh, qwk, qwx, qb,
             q_out, k_out, xk_out):
    f32 = jnp.float32
    W = w[:, :]
    X0 = x0[:, :]
    sa_d = sad[:, :]
    sa_s = sas[:, :]

    def dot(a, b):
        return jnp.dot(a, b, preferred_element_type=f32)

    def tdot(a, b):
        return lax.dot_general(a, b, (((0,), (0,)), ((), ())),
                               preferred_element_type=f32)

    ones_col = jnp.ones((_N, 1), f32)
    deg_d = dot(W, ones_col)
    deg_s = tdot(W, ones_col)

    def rep(Gfi, Gfj, Gbi, Gbj, q0f, waf, q0b, wab, AnT, bn, gmwT, gmb, fmwT, fmb):
        R = (deg_d * q0f[:, :] + sa_d * waf[:, :]
             + deg_s * q0b[:, :] + sa_s * wab[:, :])
        X = X0
        for _ in range(_PROP):
            X = (deg_d * dot(X, Gfi[:, :]) + deg_s * dot(X, Gbi[:, :])
                 + dot(dot(W, X), Gfj[:, :]) + dot(tdot(W, X), Gbj[:, :]) + R)
        h = dot(X, AnT[:, :]) + bn[:, :]
        g = _sigmoid(dot(h, gmwT[:, :]) + gmb[:, :])
        hv = dot(h, fmwT[:, :]) + fmb[:, :]
        return jnp.sum(g * hv, axis=0, keepdims=True)

    h_av = rep(a_Gfi, a_Gfj, a_Gbi, a_Gbj, a_q0f, a_waf, a_q0b, a_wab,
               a_AnT, a_bn, a_gmwT, a_gmb, a_fmwT, a_fmb)
    h_pol = rep(p_Gfi, p_Gfj, p_Gbi, p_Gbj, p_q0f, p_waf, p_q0b, p_wab,
                p_AnT, p_bn, p_gmwT, p_gmb, p_fmwT, p_fmb)

    kint = lax.broadcasted_iota(jnp.int32, (16, 1), 0)
    kcol = kint.astype(f32)
    row0 = dot(h_pol, pwat[:, :]) + pbr[:, :]
    z = 0.1 * (row0 + kcol * pwk[:, :])
    xk = (jnp.maximum(z, 0.0) + jnp.log(1.0 + jnp.exp(-jnp.abs(z)))) * 10.0
    q_base = dot(h_av, qwh[:, :])[0, 0] + qb[0, 0]
    Q = q_base + kcol * qwk[0, 0] + dot(xk, qwx[:, :])
    Qm = jnp.where(kint < _M1, Q, -1e30)
    qmax = jnp.max(Qm, keepdims=True)
    kstar = jnp.min(jnp.where(Qm == qmax, kint, 2**30), keepdims=True)
    q_out[:, :] = qmax
    k_out[:, :] = kstar
    sel = (kint == kstar).astype(f32)
    xk_out[:, :] = jnp.sum(xk * sel, axis=0, keepdims=True)


def _tc_forward(X0, W, sad, sas, fa, fp, head, interpret=False):
    out = pl.pallas_call(
        _tc_body,
        out_shape=(
            jax.ShapeDtypeStruct((1, 1), jnp.float32),
            jax.ShapeDtypeStruct((1, 1), jnp.int32),
            jax.ShapeDtypeStruct((1, 128), jnp.float32),
        ),
        interpret=interpret,
    )(X0, W, sad, sas, *fa, *fp, *head)
    return out


def _fold_head(params):
    pw = params["pw"].reshape(128, 51)
    qw = params["qw"][0]
    return (pw[:, :50].T, pw[:, 50][None, :], params["pb"].reshape(1, 128),
            qw[:50][:, None], qw[50].reshape(1, 1), qw[51:][:, None],
            params["qb"].reshape(1, 1))


def kernel(x, edge_attr, params, edge_index, done):
    src = edge_index[0]
    dst = edge_index[1]
    zeros = jnp.zeros((1024, 16), jnp.float32)
    X0 = x.reshape(_N, _NVEC)
    fa, fp = _fold_all(params)
    head = _fold_head(params)
    Wr, sad_r, sas_r = _sc_build()(src, dst, edge_attr, zeros)
    W = Wr.reshape(1024, 1024)
    sad = sad_r.reshape(1024, 1)
    sas = sas_r.reshape(1024, 1)
    q, k, xkrow = _tc_forward(X0, W, sad, sas, fa, fp, head)
    return q.reshape(()), k.reshape(()), xkrow.reshape(8, 16)
